# baseline (speedup 1.0000x reference)
import functools

import jax
import jax.numpy as jnp
from jax import lax
from jax.experimental import pallas as pl
from jax.experimental.pallas import tpu as pltpu
from jax.experimental.pallas import tpu_sc as plsc

_INT_MIN = -(2 ** 31)
_LANES = 16


def _cos_body(emb_ref, cos_ref):
    f32 = jnp.float32
    emb = emb_ref[...]
    gram = lax.dot_general(emb, emb, (((1,), (1,)), ((), ())),
                           preferred_element_type=f32)
    nrm = jnp.sqrt(jnp.sum(emb * emb, axis=1))
    cos_ref[...] = gram / (nrm[:, None] * nrm[None, :] + 1e-8)


def _topk_mask_sc_body(topk, n, rows_per_worker, cos_hbm, t_hbm,
                       block_v, keys_v, out_v):
    nvec = n // _LANES
    k_splat = jnp.full((_LANES,), topk, jnp.int32)
    int_min = jnp.full((_LANES,), _INT_MIN, jnp.int32)
    zero = jnp.zeros((_LANES,), jnp.int32)

    info = plsc.get_sparse_core_info()
    wid = lax.axis_index("s") * info.num_cores + lax.axis_index("c")
    base = wid * rows_per_worker

    pltpu.sync_copy(cos_hbm.at[pl.ds(base, rows_per_worker)], block_v)

    for r in range(rows_per_worker):
        for k in range(nvec):
            x = block_v[r, pl.ds(k * _LANES, _LANES)]
            b = lax.bitcast_convert_type(x, jnp.int32)
            ks = jnp.where(b >= 0, b, jnp.bitwise_xor(jnp.bitwise_not(b), int_min))
            keys_v[r, pl.ds(k * _LANES, _LANES)] = ks

    tss = []
    for r in range(rows_per_worker):
        keys = tuple(keys_v[r, pl.ds(k * _LANES, _LANES)] for k in range(nvec))

        def bit_body(_, c):
            tu, m, keys = c
            trial_u = tu | m
            trial_s = trial_u ^ int_min
            cnt = zero
            for k in range(nvec):
                cnt = cnt + plsc.all_reduce_population_count(keys[k] >= trial_s)
            tu = jnp.where(cnt >= k_splat, trial_u, tu)
            return tu, lax.shift_right_logical(m, 1), keys

        tu, _, _ = lax.fori_loop(0, 32, bit_body, (zero, int_min, keys))
        tss.append(tu ^ int_min)

    for r in range(rows_per_worker):
        ts = tss[r]
        n_gt = zero
        for k in range(nvec):
            ks = keys_v[r, pl.ds(k * _LANES, _LANES)]
            n_gt = n_gt + plsc.all_reduce_population_count(ks > ts)
        r_take = k_splat - n_gt
        c_eq = zero
        for k in range(nvec):
            ks = keys_v[r, pl.ds(k * _LANES, _LANES)]
            eq = ks == ts
            pc = plsc.cumsum(eq.astype(jnp.int32)) + c_eq
            keep = (ks > ts) | (eq & (pc <= r_take))
            out_v[r, pl.ds(k * _LANES, _LANES)] = jnp.where(keep, 1.0, 0.0)
            c_eq = c_eq + plsc.all_reduce_population_count(eq)
    pltpu.sync_copy(out_v, t_hbm.at[pl.ds(base, rows_per_worker)])


def _topk_mask_sc(cos, topk):
    n = cos.shape[0]
    info = plsc.get_sparse_core_info()
    nworkers = info.num_cores * info.num_subcores
    rows_per_worker = n // nworkers
    mesh = plsc.VectorSubcoreMesh(core_axis_name="c", subcore_axis_name="s")
    body = functools.partial(_topk_mask_sc_body, topk, n, rows_per_worker)
    fn = pl.kernel(
        body,
        out_type=jax.ShapeDtypeStruct((n, n), jnp.float32),
        mesh=mesh,
        compiler_params=pltpu.CompilerParams(needs_layout_passes=False,
                                             use_tc_tiling_on_sc=True),
        scratch_types=[
            pltpu.VMEM((rows_per_worker, n), jnp.float32),
            pltpu.VMEM((rows_per_worker, n), jnp.int32),
            pltpu.VMEM((rows_per_worker, n), jnp.float32),
        ],
    )
    return fn(cos)


def _wx_body(x_ref, w_ref, wx_ref):
    f32 = jnp.float32
    w = w_ref[...]
    for b in range(x_ref.shape[0]):
        wx_ref[b] = lax.dot_general(w, x_ref[b], (((0,), (0,)), ((), ())),
                                    preferred_element_type=f32)


def _amat_body(t_ref, a_ref):
    f32 = jnp.float32
    n = t_ref.shape[0]
    t_mask = t_ref[...]
    sym = jnp.minimum(t_mask + t_mask.T, 1.0)
    ii = lax.broadcasted_iota(jnp.int32, (n, n), 0)
    jj = lax.broadcasted_iota(jnp.int32, (n, n), 1)
    eye = (ii == jj).astype(f32)
    nbr = jnp.minimum(sym + eye, 1.0)
    common = lax.dot_general(nbr, nbr, (((1,), (1,)), ((), ())),
                             preferred_element_type=f32)
    maxc = jnp.max(common)
    edge_mask = sym * (common > 1.0).astype(f32)
    coeff = jnp.where(edge_mask > 0, common * common / maxc, 0.0)
    tc = t_mask * coeff
    deg = jnp.sum(tc, axis=1)
    dinv = jnp.where(deg > 0, 1.0 / jnp.sqrt(deg), 0.0)
    a_ref[...] = tc * (dinv[:, None] * dinv[None, :])


def _propagate_body(a_ref, x_ref, w_ref, b_ref, y_ref):
    f32 = jnp.float32
    wx = lax.dot_general(w_ref[...], x_ref[0], (((0,), (0,)), ((), ())),
                         preferred_element_type=f32)
    yb = lax.dot_general(wx, a_ref[...], (((1,), (1,)), ((), ())),
                         preferred_element_type=f32)
    y_ref[0] = yb + b_ref[...]


def _final_fused_body(t_ref, x_ref, w_ref, b_ref, y_ref):
    f32 = jnp.float32
    n = t_ref.shape[0]
    t_mask = t_ref[...]
    sym = jnp.minimum(t_mask + t_mask.T, 1.0)
    ii = lax.broadcasted_iota(jnp.int32, (n, n), 0)
    jj = lax.broadcasted_iota(jnp.int32, (n, n), 1)
    eye = (ii == jj).astype(f32)
    nbr = jnp.minimum(sym + eye, 1.0)
    common = lax.dot_general(nbr, nbr, (((1,), (1,)), ((), ())),
                             preferred_element_type=f32)
    maxc = jnp.max(common)
    edge_mask = sym * (common > 1.0).astype(f32)
    coeff = jnp.where(edge_mask > 0, common * common / maxc, 0.0)
    tc = t_mask * coeff
    deg = jnp.sum(tc, axis=1)
    dinv = jnp.where(deg > 0, 1.0 / jnp.sqrt(deg), 0.0)
    a_mat = tc * (dinv[:, None] * dinv[None, :])
    w = w_ref[...]
    bias_col = b_ref[...]
    for b in range(x_ref.shape[0]):
        wx = lax.dot_general(w, x_ref[b], (((0,), (0,)), ((), ())),
                             preferred_element_type=f32)
        yb = lax.dot_general(wx, a_mat, (((1,), (1,)), ((), ())),
                             preferred_element_type=f32)
        y_ref[b] = yb + bias_col


def _final_body(t_ref, wx_ref, b_ref, y_ref):
    f32 = jnp.float32
    n = t_ref.shape[0]
    t_mask = t_ref[...]
    sym = jnp.minimum(t_mask + t_mask.T, 1.0)
    ii = lax.broadcasted_iota(jnp.int32, (n, n), 0)
    jj = lax.broadcasted_iota(jnp.int32, (n, n), 1)
    eye = (ii == jj).astype(f32)
    nbr = jnp.minimum(sym + eye, 1.0)
    common = lax.dot_general(nbr, nbr, (((1,), (1,)), ((), ())),
                             preferred_element_type=f32)
    maxc = jnp.max(common)
    edge_mask = sym * (common > 1.0).astype(f32)
    coeff = jnp.where(edge_mask > 0, common * common / maxc, 0.0)
    tc = t_mask * coeff
    deg = jnp.sum(tc, axis=1)
    dinv = jnp.where(deg > 0, 1.0 / jnp.sqrt(deg), 0.0)
    a_mat = tc * (dinv[:, None] * dinv[None, :])
    bias_col = b_ref[...]
    for b in range(wx_ref.shape[0]):
        yb = lax.dot_general(wx_ref[b], a_mat, (((1,), (1,)), ((), ())),
                             preferred_element_type=f32)
        y_ref[b] = yb + bias_col


def kernel(x, weight, bias, embedding):
    batch, seq, n = x.shape
    topk = int(0.3 * n)
    f32 = jnp.float32

    cos = pl.pallas_call(
        _cos_body,
        out_shape=jax.ShapeDtypeStruct((n, n), f32),
    )(embedding)

    t_mask = _topk_mask_sc(cos, topk)

    wx = pl.pallas_call(
        _wx_body,
        out_shape=jax.ShapeDtypeStruct((batch, seq, n), f32),
    )(x, weight)

    y = pl.pallas_call(
        _final_body,
        out_shape=jax.ShapeDtypeStruct((batch, seq, n), f32),
    )(t_mask, wx, bias.reshape(seq, 1))
    return y


# speedup vs baseline: 1.0169x; 1.0169x over previous
import functools

import jax
import jax.numpy as jnp
from jax import lax
from jax.experimental import pallas as pl
from jax.experimental.pallas import tpu as pltpu
from jax.experimental.pallas import tpu_sc as plsc

_INT_MIN = -(2 ** 31)
_LANES = 16


def _cos_body(emb_ref, cos_ref):
    f32 = jnp.float32
    emb = emb_ref[...]
    gram = lax.dot_general(emb, emb, (((1,), (1,)), ((), ())),
                           preferred_element_type=f32)
    nrm = jnp.sqrt(jnp.sum(emb * emb, axis=1))
    cos_ref[...] = gram / (nrm[:, None] * nrm[None, :] + 1e-8)


def _topk_mask_sc_body(topk, n, rows_per_worker, cos_hbm, t_hbm,
                       block_v, keys_v, out_v):
    nvec = n // _LANES
    k_splat = jnp.full((_LANES,), topk, jnp.int32)
    int_min = jnp.full((_LANES,), _INT_MIN, jnp.int32)
    zero = jnp.zeros((_LANES,), jnp.int32)

    info = plsc.get_sparse_core_info()
    wid = lax.axis_index("s") * info.num_cores + lax.axis_index("c")
    base = wid * rows_per_worker

    pltpu.sync_copy(cos_hbm.at[pl.ds(base, rows_per_worker)], block_v)

    for r in range(rows_per_worker):
        for k in range(nvec):
            x = block_v[r, pl.ds(k * _LANES, _LANES)]
            b = lax.bitcast_convert_type(x, jnp.int32)
            ks = jnp.where(b >= 0, b, jnp.bitwise_xor(jnp.bitwise_not(b), int_min))
            keys_v[r, pl.ds(k * _LANES, _LANES)] = ks

    tss = [None] * rows_per_worker
    for r in range(0, rows_per_worker, 2):
        keys_a = tuple(keys_v[r, pl.ds(k * _LANES, _LANES)] for k in range(nvec))
        keys_b = tuple(keys_v[r + 1, pl.ds(k * _LANES, _LANES)] for k in range(nvec))

        def bit_body(_, c):
            tu_a, tu_b, m, keys_a, keys_b = c
            trial_a = tu_a | m
            trial_b = tu_b | m
            trial_as = trial_a ^ int_min
            trial_bs = trial_b ^ int_min
            cnt_a = zero
            cnt_b = zero
            for k in range(nvec):
                cnt_a = cnt_a + plsc.all_reduce_population_count(keys_a[k] >= trial_as)
                cnt_b = cnt_b + plsc.all_reduce_population_count(keys_b[k] >= trial_bs)
            tu_a = jnp.where(cnt_a >= k_splat, trial_a, tu_a)
            tu_b = jnp.where(cnt_b >= k_splat, trial_b, tu_b)
            return tu_a, tu_b, lax.shift_right_logical(m, 1), keys_a, keys_b

        tu_a, tu_b, _, _, _ = lax.fori_loop(
            0, 32, bit_body, (zero, zero, int_min, keys_a, keys_b))
        tss[r] = tu_a ^ int_min
        tss[r + 1] = tu_b ^ int_min

    for r in range(rows_per_worker):
        ts = tss[r]
        n_gt = zero
        for k in range(nvec):
            ks = keys_v[r, pl.ds(k * _LANES, _LANES)]
            n_gt = n_gt + plsc.all_reduce_population_count(ks > ts)
        r_take = k_splat - n_gt
        c_eq = zero
        for k in range(nvec):
            ks = keys_v[r, pl.ds(k * _LANES, _LANES)]
            eq = ks == ts
            pc = plsc.cumsum(eq.astype(jnp.int32)) + c_eq
            keep = (ks > ts) | (eq & (pc <= r_take))
            out_v[r, pl.ds(k * _LANES, _LANES)] = jnp.where(keep, 1.0, 0.0)
            c_eq = c_eq + plsc.all_reduce_population_count(eq)
    pltpu.sync_copy(out_v, t_hbm.at[pl.ds(base, rows_per_worker)])


def _topk_mask_sc(cos, topk):
    n = cos.shape[0]
    info = plsc.get_sparse_core_info()
    nworkers = info.num_cores * info.num_subcores
    rows_per_worker = n // nworkers
    mesh = plsc.VectorSubcoreMesh(core_axis_name="c", subcore_axis_name="s")
    body = functools.partial(_topk_mask_sc_body, topk, n, rows_per_worker)
    fn = pl.kernel(
        body,
        out_type=jax.ShapeDtypeStruct((n, n), jnp.float32),
        mesh=mesh,
        compiler_params=pltpu.CompilerParams(needs_layout_passes=False),
        scratch_types=[
            pltpu.VMEM((rows_per_worker, n), jnp.float32),
            pltpu.VMEM((rows_per_worker, n), jnp.int32),
            pltpu.VMEM((rows_per_worker, n), jnp.float32),
        ],
    )
    return fn(cos)


def _wx_body(x_ref, w_ref, wx_ref):
    f32 = jnp.float32
    w = w_ref[...]
    for b in range(x_ref.shape[0]):
        wx_ref[b] = lax.dot_general(w, x_ref[b], (((0,), (0,)), ((), ())),
                                    preferred_element_type=f32)


def _amat_body(t_ref, a_ref):
    f32 = jnp.float32
    n = t_ref.shape[0]
    t_mask = t_ref[...]
    sym = jnp.minimum(t_mask + t_mask.T, 1.0)
    ii = lax.broadcasted_iota(jnp.int32, (n, n), 0)
    jj = lax.broadcasted_iota(jnp.int32, (n, n), 1)
    eye = (ii == jj).astype(f32)
    nbr = jnp.minimum(sym + eye, 1.0)
    common = lax.dot_general(nbr, nbr, (((1,), (1,)), ((), ())),
                             preferred_element_type=f32)
    maxc = jnp.max(common)
    edge_mask = sym * (common > 1.0).astype(f32)
    coeff = jnp.where(edge_mask > 0, common * common / maxc, 0.0)
    tc = t_mask * coeff
    deg = jnp.sum(tc, axis=1)
    dinv = jnp.where(deg > 0, 1.0 / jnp.sqrt(deg), 0.0)
    a_ref[...] = tc * (dinv[:, None] * dinv[None, :])


def _propagate_body(a_ref, x_ref, w_ref, b_ref, y_ref):
    f32 = jnp.float32
    wx = lax.dot_general(w_ref[...], x_ref[0], (((0,), (0,)), ((), ())),
                         preferred_element_type=f32)
    yb = lax.dot_general(wx, a_ref[...], (((1,), (1,)), ((), ())),
                         preferred_element_type=f32)
    y_ref[0] = yb + b_ref[...]


def _final_fused_body(t_ref, x_ref, w_ref, b_ref, y_ref):
    f32 = jnp.float32
    n = t_ref.shape[0]
    t_mask = t_ref[...]
    sym = jnp.minimum(t_mask + t_mask.T, 1.0)
    ii = lax.broadcasted_iota(jnp.int32, (n, n), 0)
    jj = lax.broadcasted_iota(jnp.int32, (n, n), 1)
    eye = (ii == jj).astype(f32)
    nbr = jnp.minimum(sym + eye, 1.0)
    common = lax.dot_general(nbr, nbr, (((1,), (1,)), ((), ())),
                             preferred_element_type=f32)
    maxc = jnp.max(common)
    edge_mask = sym * (common > 1.0).astype(f32)
    coeff = jnp.where(edge_mask > 0, common * common / maxc, 0.0)
    tc = t_mask * coeff
    deg = jnp.sum(tc, axis=1)
    dinv = jnp.where(deg > 0, 1.0 / jnp.sqrt(deg), 0.0)
    a_mat = tc * (dinv[:, None] * dinv[None, :])
    w = w_ref[...]
    bias_col = b_ref[...]
    for b in range(x_ref.shape[0]):
        wx = lax.dot_general(w, x_ref[b], (((0,), (0,)), ((), ())),
                             preferred_element_type=f32)
        yb = lax.dot_general(wx, a_mat, (((1,), (1,)), ((), ())),
                             preferred_element_type=f32)
        y_ref[b] = yb + bias_col


def _final_body(t_ref, wx_ref, b_ref, y_ref):
    f32 = jnp.float32
    n = t_ref.shape[0]
    t_mask = t_ref[...]
    sym = jnp.minimum(t_mask + t_mask.T, 1.0)
    ii = lax.broadcasted_iota(jnp.int32, (n, n), 0)
    jj = lax.broadcasted_iota(jnp.int32, (n, n), 1)
    eye = (ii == jj).astype(f32)
    nbr = jnp.minimum(sym + eye, 1.0)
    common = lax.dot_general(nbr, nbr, (((1,), (1,)), ((), ())),
                             preferred_element_type=f32)
    maxc = jnp.max(common)
    edge_mask = sym * (common > 1.0).astype(f32)
    coeff = jnp.where(edge_mask > 0, common * common / maxc, 0.0)
    tc = t_mask * coeff
    deg = jnp.sum(tc, axis=1)
    dinv = jnp.where(deg > 0, 1.0 / jnp.sqrt(deg), 0.0)
    a_mat = tc * (dinv[:, None] * dinv[None, :])
    bias_col = b_ref[...]
    for b in range(wx_ref.shape[0]):
        yb = lax.dot_general(wx_ref[b], a_mat, (((1,), (1,)), ((), ())),
                             preferred_element_type=f32)
        y_ref[b] = yb + bias_col


def kernel(x, weight, bias, embedding):
    batch, seq, n = x.shape
    topk = int(0.3 * n)
    f32 = jnp.float32

    cos = pl.pallas_call(
        _cos_body,
        out_shape=jax.ShapeDtypeStruct((n, n), f32),
    )(embedding)

    t_mask = _topk_mask_sc(cos, topk)

    wx = pl.pallas_call(
        _wx_body,
        out_shape=jax.ShapeDtypeStruct((batch, seq, n), f32),
    )(x, weight)

    y = pl.pallas_call(
        _final_body,
        out_shape=jax.ShapeDtypeStruct((batch, seq, n), f32),
    )(t_mask, wx, bias.reshape(seq, 1))
    return y
